# baseline (speedup 1.0000x reference)
import functools

import jax
import jax.numpy as jnp
from jax import lax
from jax.experimental import pallas as pl
from jax.experimental.pallas import tpu as pltpu
from jax.experimental.pallas import tpu_sc as plsc

_K = 16
_TH = 0.5
_INF = float("inf")


def _bf_round(x):
    u = plsc.bitcast(x, jnp.uint32)
    r = (u + jnp.uint32(0x7FFF) + ((u >> 16) & jnp.uint32(1))) & jnp.uint32(
        0xFFFF0000
    )
    return plsc.bitcast(r, jnp.float32)


def _splat(v, lane):
    idx = jnp.full((16,), lane, jnp.int32)
    return v.at[idx].get(mode="promise_in_bounds", unique_indices=False)


def _make_sc_knn(b, n):
    rows_per_w = (b * n) // 32
    groups = rows_per_w // 16
    chunks = n // 16
    wpb = 32 // b
    mesh = plsc.VectorSubcoreMesh(core_axis_name="c", subcore_axis_name="s")

    @functools.partial(
        pl.kernel,
        out_type=jax.ShapeDtypeStruct((b * n * _K,), jnp.int32),
        mesh=mesh,
        compiler_params=pltpu.CompilerParams(needs_layout_passes=False),
        scratch_types=[
            pltpu.VMEM((n,), jnp.float32),
            pltpu.VMEM((n,), jnp.float32),
            pltpu.VMEM((n,), jnp.float32),
            pltpu.VMEM((n,), jnp.float32),
            pltpu.VMEM((n,), jnp.float32),
            pltpu.VMEM((n,), jnp.float32),
            pltpu.VMEM((n,), jnp.float32),
            pltpu.VMEM((96,), jnp.float32),
            pltpu.VMEM((96,), jnp.int32),
            pltpu.VMEM((rows_per_w * _K,), jnp.int32),
        ],
    )
    def sc_knn(xa, ya, za, out_hbm, xv, yv, zv, xm2, ym2, zm2, csqv, bufd, bufi, obuf):
        wid = lax.axis_index("s") * 2 + lax.axis_index("c")
        batch = wid // wpb
        r0 = (wid % wpb) * rows_per_w

        pltpu.sync_copy(xa.at[pl.ds(batch * n, n)], xv)
        pltpu.sync_copy(ya.at[pl.ds(batch * n, n)], yv)
        pltpu.sync_copy(za.at[pl.ds(batch * n, n)], zv)

        def prep(i, carry):
            s = pl.ds(i * 16, 16)
            cx = xv[s]
            cy = yv[s]
            cz = zv[s]
            xm2[s] = _bf_round(cx) * -2.0
            ym2[s] = _bf_round(cy) * -2.0
            zm2[s] = _bf_round(cz) * -2.0
            csqv[s] = cx * cx + cy * cy + cz * cz
            return carry

        lax.fori_loop(0, chunks, prep, 0)

        lanes = lax.iota(jnp.int32, 16)

        def flush(top_v, top_i, pos):
            for blk in range(5):
                bd = bufd[pl.ds(blk * 16, 16)]
                bi = bufi[pl.ds(blk * 16, 16)]
                valid = (lanes + blk * 16) < pos
                bd = jnp.where(valid, bd, _INF)
                sd, si = plsc.sort_key_val(bd, bi, descending=True)
                take_new = sd < top_v
                nv = jnp.where(take_new, sd, top_v)
                ni = jnp.where(take_new, si, top_i)
                top_v, top_i = plsc.sort_key_val(nv, ni)
            bound = _splat(top_v, 15)
            return top_v, top_i, bound

        def group_body(g, carry):
            qs = pl.ds(r0 + g * 16, 16)
            qx = xv[qs]
            qy = yv[qs]
            qz = zv[qs]
            qbx = _bf_round(qx)
            qby = _bf_round(qy)
            qbz = _bf_round(qz)
            qsq = qx * qx + qy * qy + qz * qz

            def row_body(l, carry2):
                bx = _splat(qbx, l)
                by = _splat(qby, l)
                bz = _splat(qbz, l)
                sq = _splat(qsq, l)

                def chunk_body(i, st):
                    top_v, top_i, bound, pos = st
                    s = pl.ds(i * 16, 16)
                    d = (sq + ((bx * xm2[s] + by * ym2[s]) + bz * zm2[s])) + csqv[s]
                    de = jnp.where(d > _TH, _INF, d)
                    m = de < bound
                    cnt = plsc.all_reduce_population_count(m)[0]

                    def append(st2):
                        top_v, top_i, bound, pos = st2
                        iv = lanes + i * 16
                        plsc.store_compressed(bufd.at[pl.ds(pos, 16)], de, mask=m)
                        plsc.store_compressed(bufi.at[pl.ds(pos, 16)], iv, mask=m)
                        pos2 = pos + cnt

                        def do_flush(st3):
                            tv, ti, _, p = st3
                            tv, ti, bnd = flush(tv, ti, p)
                            return tv, ti, bnd, jnp.int32(0)

                        return lax.cond(
                            pos2 >= 64,
                            do_flush,
                            lambda st3: st3,
                            (top_v, top_i, bound, pos2),
                        )

                    return lax.cond(cnt > 0, append, lambda st2: st2, st)

                init = (
                    jnp.full((16,), _INF, jnp.float32),
                    lanes,
                    jnp.full((16,), _INF, jnp.float32),
                    jnp.int32(0),
                )
                top_v, top_i, bound, pos = lax.fori_loop(0, chunks, chunk_body, init)
                top_v, top_i, bound = flush(top_v, top_i, pos)
                obuf[pl.ds((g * 16 + l) * _K, _K)] = top_i
                return carry2

            lax.fori_loop(0, 16, row_body, 0)
            return carry

        lax.fori_loop(0, groups, group_body, 0)
        pltpu.sync_copy(
            obuf, out_hbm.at[pl.ds((batch * n + r0) * _K, rows_per_w * _K)]
        )

    return sc_knn


def kernel(coords):
    b, n, d = coords.shape
    xa = coords[:, :, 0].reshape(-1)
    ya = coords[:, :, 1].reshape(-1)
    za = coords[:, :, 2].reshape(-1)
    nn_idx = _make_sc_knn(b, n)(xa, ya, za).reshape(b, n, _K)
    center_idx = jnp.broadcast_to(
        jnp.arange(n, dtype=nn_idx.dtype)[None, :, None], (b, n, _K)
    )
    return jnp.stack((nn_idx, center_idx), axis=0)


# speedup vs baseline: 3.7772x; 3.7772x over previous
import functools

import jax
import jax.numpy as jnp
from jax import lax
from jax.experimental import pallas as pl
from jax.experimental.pallas import tpu as pltpu
from jax.experimental.pallas import tpu_sc as plsc

_K = 16
_TH = 0.5
_INF = float("inf")


def _bf_round(x):
    u = plsc.bitcast(x, jnp.uint32)
    r = (u + jnp.uint32(0x7FFF) + ((u >> 16) & jnp.uint32(1))) & jnp.uint32(
        0xFFFF0000
    )
    return plsc.bitcast(r, jnp.float32)


def _splat(v, lane):
    idx = jnp.full((16,), lane, jnp.int32)
    return v.at[idx].get(mode="promise_in_bounds", unique_indices=False)


def _make_sc_knn(b, n):
    rows_per_w = (b * n) // 32
    groups = rows_per_w // 16
    chunks = n // 16
    wpb = 32 // b
    mesh = plsc.VectorSubcoreMesh(core_axis_name="c", subcore_axis_name="s")

    @functools.partial(
        pl.kernel,
        out_type=jax.ShapeDtypeStruct((b * n * _K,), jnp.int32),
        mesh=mesh,
        compiler_params=pltpu.CompilerParams(needs_layout_passes=False),
        scratch_types=[
            pltpu.VMEM((n,), jnp.float32),
            pltpu.VMEM((n,), jnp.float32),
            pltpu.VMEM((n,), jnp.float32),
            pltpu.VMEM((n,), jnp.float32),
            pltpu.VMEM((n,), jnp.float32),
            pltpu.VMEM((n,), jnp.float32),
            pltpu.VMEM((n,), jnp.float32),
            pltpu.VMEM((160,), jnp.float32),
            pltpu.VMEM((160,), jnp.int32),
            pltpu.VMEM((rows_per_w * _K,), jnp.int32),
        ],
    )
    def sc_knn(xa, ya, za, out_hbm, xv, yv, zv, xm2, ym2, zm2, csqv, bufd, bufi, obuf):
        wid = lax.axis_index("s") * 2 + lax.axis_index("c")
        batch = wid // wpb
        r0 = (wid % wpb) * rows_per_w

        pltpu.sync_copy(xa.at[pl.ds(batch * n, n)], xv)
        pltpu.sync_copy(ya.at[pl.ds(batch * n, n)], yv)
        pltpu.sync_copy(za.at[pl.ds(batch * n, n)], zv)

        def prep(i, carry):
            s = pl.ds(i * 16, 16)
            cx = xv[s]
            cy = yv[s]
            cz = zv[s]
            xm2[s] = _bf_round(cx) * -2.0
            ym2[s] = _bf_round(cy) * -2.0
            zm2[s] = _bf_round(cz) * -2.0
            csqv[s] = cx * cx + cy * cy + cz * cz
            return carry

        lax.fori_loop(0, chunks, prep, 0)

        lanes = lax.iota(jnp.int32, 16)

        def merge2(av, ai, bv, bi):
            rv = lax.rev(bv, (0,))
            ri = lax.rev(bi, (0,))
            take_b = rv < av
            nv = jnp.where(take_b, rv, av)
            ni = jnp.where(take_b, ri, ai)
            return plsc.sort_key_val(nv, ni)

        def flush(top_v, top_i, pos):
            blocks = []
            for blk in range(8):
                bd = bufd[pl.ds(blk * 16, 16)]
                bi = bufi[pl.ds(blk * 16, 16)]
                valid = (lanes + blk * 16) < pos
                bd = jnp.where(valid, bd, _INF)
                blocks.append(plsc.sort_key_val(bd, bi))
            while len(blocks) > 1:
                nxt = []
                for a in range(0, len(blocks), 2):
                    nxt.append(merge2(*blocks[a], *blocks[a + 1]))
                blocks = nxt
            top_v, top_i = merge2(top_v, top_i, *blocks[0])
            bound = _splat(top_v, 15)
            return top_v, top_i, bound

        def group_body(g, carry):
            qs = pl.ds(r0 + g * 16, 16)
            qx = xv[qs]
            qy = yv[qs]
            qz = zv[qs]
            qbx = _bf_round(qx)
            qby = _bf_round(qy)
            qbz = _bf_round(qz)
            qsq = qx * qx + qy * qy + qz * qz

            def row_body(l, carry2):
                bx = _splat(qbx, l)
                by = _splat(qby, l)
                bz = _splat(qbz, l)
                sq = _splat(qsq, l)

                def chunk_body(j, st):
                    top_v, top_i, bound, pos = st
                    base = j * 64
                    des, ms = [], []
                    any_m = None
                    for u in range(4):
                        s = pl.ds(base + u * 16, 16)
                        d = (
                            sq + ((bx * xm2[s] + by * ym2[s]) + bz * zm2[s])
                        ) + csqv[s]
                        de = jnp.where(d > _TH, _INF, d)
                        m = de < bound
                        des.append(de)
                        ms.append(m)
                        any_m = m if any_m is None else (any_m | m)
                    anyc = plsc.all_reduce_population_count(any_m)[0]

                    def append(st2):
                        top_v, top_i, bound, pos = st2
                        for u in range(4):
                            cu = plsc.all_reduce_population_count(ms[u])[0]
                            plsc.store_compressed(
                                bufd.at[pl.ds(pos, 16)], des[u], mask=ms[u]
                            )
                            plsc.store_compressed(
                                bufi.at[pl.ds(pos, 16)],
                                lanes + (base + u * 16),
                                mask=ms[u],
                            )
                            pos = pos + cu

                        def do_flush(st3):
                            tv, ti, _, p = st3
                            tv, ti, bnd = flush(tv, ti, p)
                            return tv, ti, bnd, jnp.int32(0)

                        return lax.cond(
                            pos >= 64,
                            do_flush,
                            lambda st3: st3,
                            (top_v, top_i, bound, pos),
                        )

                    return lax.cond(anyc > 0, append, lambda st2: st2, st)

                init = (
                    jnp.full((16,), _INF, jnp.float32),
                    lanes,
                    jnp.full((16,), _INF, jnp.float32),
                    jnp.int32(0),
                )
                top_v, top_i, bound, pos = lax.fori_loop(
                    0, chunks // 4, chunk_body, init
                )
                top_v, top_i, bound = flush(top_v, top_i, pos)
                obuf[pl.ds((g * 16 + l) * _K, _K)] = top_i
                return carry2

            lax.fori_loop(0, 16, row_body, 0)
            return carry

        lax.fori_loop(0, groups, group_body, 0)
        pltpu.sync_copy(
            obuf, out_hbm.at[pl.ds((batch * n + r0) * _K, rows_per_w * _K)]
        )

    return sc_knn


def kernel(coords):
    b, n, d = coords.shape
    xa = coords[:, :, 0].reshape(-1)
    ya = coords[:, :, 1].reshape(-1)
    za = coords[:, :, 2].reshape(-1)
    nn_idx = _make_sc_knn(b, n)(xa, ya, za).reshape(b, n, _K)
    center_idx = jnp.broadcast_to(
        jnp.arange(n, dtype=nn_idx.dtype)[None, :, None], (b, n, _K)
    )
    return jnp.stack((nn_idx, center_idx), axis=0)


# speedup vs baseline: 4.5455x; 1.2034x over previous
import functools

import jax
import jax.numpy as jnp
from jax import lax
from jax.experimental import pallas as pl
from jax.experimental.pallas import tpu as pltpu
from jax.experimental.pallas import tpu_sc as plsc

_K = 16
_TH = 0.5
_INF = float("inf")


def _bf_round(x):
    u = plsc.bitcast(x, jnp.uint32)
    r = (u + jnp.uint32(0x7FFF) + ((u >> 16) & jnp.uint32(1))) & jnp.uint32(
        0xFFFF0000
    )
    return plsc.bitcast(r, jnp.float32)


def _splat(v, lane):
    idx = jnp.full((16,), lane, jnp.int32)
    return v.at[idx].get(mode="promise_in_bounds", unique_indices=False)


def _make_sc_knn(b, n):
    rows_per_w = (b * n) // 32
    groups = rows_per_w // 16
    chunks = n // 16
    wpb = 32 // b
    mesh = plsc.VectorSubcoreMesh(core_axis_name="c", subcore_axis_name="s")

    @functools.partial(
        pl.kernel,
        out_type=jax.ShapeDtypeStruct((b * n * _K,), jnp.int32),
        mesh=mesh,
        compiler_params=pltpu.CompilerParams(needs_layout_passes=False),
        scratch_types=[
            pltpu.VMEM((n,), jnp.float32),
            pltpu.VMEM((n,), jnp.float32),
            pltpu.VMEM((n,), jnp.float32),
            pltpu.VMEM((n,), jnp.float32),
            pltpu.VMEM((n,), jnp.float32),
            pltpu.VMEM((n,), jnp.float32),
            pltpu.VMEM((n,), jnp.float32),
            pltpu.VMEM((160,), jnp.float32),
            pltpu.VMEM((160,), jnp.int32),
            pltpu.VMEM((rows_per_w * _K,), jnp.int32),
        ],
    )
    def sc_knn(xa, ya, za, out_hbm, xv, yv, zv, xm2, ym2, zm2, csqv, bufd, bufi, obuf):
        wid = lax.axis_index("s") * 2 + lax.axis_index("c")
        batch = wid // wpb
        r0 = (wid % wpb) * rows_per_w

        pltpu.sync_copy(xa.at[pl.ds(batch * n, n)], xv)
        pltpu.sync_copy(ya.at[pl.ds(batch * n, n)], yv)
        pltpu.sync_copy(za.at[pl.ds(batch * n, n)], zv)

        def prep(i, carry):
            s = pl.ds(i * 16, 16)
            cx = xv[s]
            cy = yv[s]
            cz = zv[s]
            xm2[s] = _bf_round(cx) * -2.0
            ym2[s] = _bf_round(cy) * -2.0
            zm2[s] = _bf_round(cz) * -2.0
            csqv[s] = cx * cx + cy * cy + cz * cz
            return carry

        lax.fori_loop(0, chunks, prep, 0)

        lanes = lax.iota(jnp.int32, 16)

        def merge2(av, ai, bv, bi):
            rv = lax.rev(bv, (0,))
            ri = lax.rev(bi, (0,))
            take_b = rv < av
            nv = jnp.where(take_b, rv, av)
            ni = jnp.where(take_b, ri, ai)
            return plsc.sort_key_val(nv, ni)

        def flush(top_v, top_i, pos):
            blocks = []
            for blk in range(8):
                bd = bufd[pl.ds(blk * 16, 16)]
                bi = bufi[pl.ds(blk * 16, 16)]
                valid = (lanes + blk * 16) < pos
                bd = jnp.where(valid, bd, _INF)
                blocks.append(plsc.sort_key_val(bd, bi))
            while len(blocks) > 1:
                nxt = []
                for a in range(0, len(blocks), 2):
                    nxt.append(merge2(*blocks[a], *blocks[a + 1]))
                blocks = nxt
            top_v, top_i = merge2(top_v, top_i, *blocks[0])
            bound = _splat(top_v, 15)
            return top_v, top_i, bound

        def group_body(g, carry):
            qs = pl.ds(r0 + g * 16, 16)
            qx = xv[qs]
            qy = yv[qs]
            qz = zv[qs]
            qbx = _bf_round(qx)
            qby = _bf_round(qy)
            qbz = _bf_round(qz)
            qsq = qx * qx + qy * qy + qz * qz

            def row_body(l, carry2):
                bx = _splat(qbx, l)
                by = _splat(qby, l)
                bz = _splat(qbz, l)
                sq = _splat(qsq, l)

                zf = jnp.zeros((16,), jnp.float32)
                zi = jnp.zeros((16,), jnp.int32)
                fmask = lanes < 0

                def do_append(st2):
                    top_v, top_i, bound, pos, pdes, pms, pbase = st2
                    for u in range(4):
                        cu = plsc.all_reduce_population_count(pms[u])[0]
                        plsc.store_compressed(
                            bufd.at[pl.ds(pos, 16)], pdes[u], mask=pms[u]
                        )
                        plsc.store_compressed(
                            bufi.at[pl.ds(pos, 16)],
                            lanes + (pbase + u * 16),
                            mask=pms[u],
                        )
                        pos = pos + cu

                    def do_flush(st3):
                        tv, ti, _, p = st3
                        tv, ti, bnd = flush(tv, ti, p)
                        return tv, ti, bnd, jnp.int32(0)

                    return lax.cond(
                        pos >= 64,
                        do_flush,
                        lambda st3: st3,
                        (top_v, top_i, bound, pos),
                    )

                def chunk_body(j, st):
                    (top_v, top_i, bound, pos, panyc, pdes, pms, pbase) = st
                    base = j * 64
                    des, ms = [], []
                    any_m = None
                    for u in range(4):
                        s = pl.ds(base + u * 16, 16)
                        d = (
                            sq + ((bx * xm2[s] + by * ym2[s]) + bz * zm2[s])
                        ) + csqv[s]
                        de = jnp.where(d > _TH, _INF, d)
                        m = de < bound
                        des.append(de)
                        ms.append(m)
                        any_m = m if any_m is None else (any_m | m)
                    anyc = plsc.all_reduce_population_count(any_m)[0]

                    top_v, top_i, bound, pos = lax.cond(
                        panyc > 0,
                        do_append,
                        lambda st2: st2[:4],
                        (top_v, top_i, bound, pos, pdes, pms, pbase),
                    )
                    return (top_v, top_i, bound, pos, anyc, tuple(des), tuple(ms), base)

                init = (
                    jnp.full((16,), _INF, jnp.float32),
                    lanes,
                    jnp.full((16,), _INF, jnp.float32),
                    jnp.int32(0),
                    jnp.int32(0),
                    (zf, zf, zf, zf),
                    (fmask, fmask, fmask, fmask),
                    jnp.int32(0),
                )
                (top_v, top_i, bound, pos, panyc, pdes, pms, pbase) = lax.fori_loop(
                    0, chunks // 4, chunk_body, init
                )
                top_v, top_i, bound, pos = lax.cond(
                    panyc > 0,
                    do_append,
                    lambda st2: st2[:4],
                    (top_v, top_i, bound, pos, pdes, pms, pbase),
                )
                top_v, top_i, bound = flush(top_v, top_i, pos)
                up = jnp.minimum(lanes + 1, 15)
                dn = jnp.maximum(lanes - 1, 0)
                nv = top_v.at[up].get(mode="promise_in_bounds")
                ni = top_i.at[up].get(mode="promise_in_bounds")
                pv = top_v.at[dn].get(mode="promise_in_bounds")
                pi = top_i.at[dn].get(mode="promise_in_bounds")
                swap_hi = (top_v == nv) & (top_i > ni)
                swap_lo = (top_v == pv) & (pi > top_i)
                top_i = jnp.where(
                    swap_hi, ni, jnp.where(swap_lo, pi, top_i)
                )
                obuf[pl.ds((g * 16 + l) * _K, _K)] = top_i
                return carry2

            lax.fori_loop(0, 16, row_body, 0)
            return carry

        lax.fori_loop(0, groups, group_body, 0)
        pltpu.sync_copy(
            obuf, out_hbm.at[pl.ds((batch * n + r0) * _K, rows_per_w * _K)]
        )

    return sc_knn


def kernel(coords):
    b, n, d = coords.shape
    xa = coords[:, :, 0].reshape(-1)
    ya = coords[:, :, 1].reshape(-1)
    za = coords[:, :, 2].reshape(-1)
    nn_idx = _make_sc_knn(b, n)(xa, ya, za).reshape(b, n, _K)
    center_idx = jnp.broadcast_to(
        jnp.arange(n, dtype=nn_idx.dtype)[None, :, None], (b, n, _K)
    )
    return jnp.stack((nn_idx, center_idx), axis=0)


# speedup vs baseline: 6.1505x; 1.3531x over previous
import functools

import jax
import jax.numpy as jnp
from jax import lax
from jax.experimental import pallas as pl
from jax.experimental.pallas import tpu as pltpu
from jax.experimental.pallas import tpu_sc as plsc

_K = 16
_TH = 0.5
_INF = float("inf")


def _bf_round(x):
    u = plsc.bitcast(x, jnp.uint32)
    r = (u + jnp.uint32(0x7FFF) + ((u >> 16) & jnp.uint32(1))) & jnp.uint32(
        0xFFFF0000
    )
    return plsc.bitcast(r, jnp.float32)


def _splat(v, lane):
    idx = jnp.full((16,), lane, jnp.int32)
    return v.at[idx].get(mode="promise_in_bounds", unique_indices=False)


def _make_sc_knn(b, n):
    rows_per_w = (b * n) // 32
    groups = rows_per_w // 16
    chunks = n // 16
    wpb = 32 // b
    mesh = plsc.VectorSubcoreMesh(core_axis_name="c", subcore_axis_name="s")

    @functools.partial(
        pl.kernel,
        out_type=jax.ShapeDtypeStruct((b * n * _K,), jnp.int32),
        mesh=mesh,
        compiler_params=pltpu.CompilerParams(needs_layout_passes=False),
        scratch_types=[
            pltpu.VMEM((n,), jnp.float32),
            pltpu.VMEM((n,), jnp.float32),
            pltpu.VMEM((n,), jnp.float32),
            pltpu.VMEM((n,), jnp.float32),
            pltpu.VMEM((n,), jnp.float32),
            pltpu.VMEM((n,), jnp.float32),
            pltpu.VMEM((n,), jnp.float32),
            pltpu.VMEM((224,), jnp.float32),
            pltpu.VMEM((224,), jnp.int32),
            pltpu.VMEM((rows_per_w * _K,), jnp.int32),
        ],
    )
    def sc_knn(xa, ya, za, out_hbm, xv, yv, zv, xm2, ym2, zm2, csqv, bufd, bufi, obuf):
        wid = lax.axis_index("s") * 2 + lax.axis_index("c")
        batch = wid // wpb
        r0 = (wid % wpb) * rows_per_w

        pltpu.sync_copy(xa.at[pl.ds(batch * n, n)], xv)
        pltpu.sync_copy(ya.at[pl.ds(batch * n, n)], yv)
        pltpu.sync_copy(za.at[pl.ds(batch * n, n)], zv)

        def prep(i, carry):
            s = pl.ds(i * 16, 16)
            cx = xv[s]
            cy = yv[s]
            cz = zv[s]
            xm2[s] = _bf_round(cx) * -2.0
            ym2[s] = _bf_round(cy) * -2.0
            zm2[s] = _bf_round(cz) * -2.0
            csqv[s] = cx * cx + cy * cy + cz * cz
            return carry

        lax.fori_loop(0, chunks, prep, 0)

        lanes = lax.iota(jnp.int32, 16)

        def merge2(av, ai, bv, bi):
            rv = lax.rev(bv, (0,))
            ri = lax.rev(bi, (0,))
            take_b = rv < av
            nv = jnp.where(take_b, rv, av)
            ni = jnp.where(take_b, ri, ai)
            return plsc.sort_key_val(nv, ni)

        def flush(top_v, top_i, pos):
            blocks = []
            for blk in range(12):
                bd = bufd[pl.ds(blk * 16, 16)]
                bi = bufi[pl.ds(blk * 16, 16)]
                valid = (lanes + blk * 16) < pos
                bd = jnp.where(valid, bd, _INF)
                blocks.append(plsc.sort_key_val(bd, bi))
            while len(blocks) > 1:
                nxt = []
                for a in range(0, len(blocks) - 1, 2):
                    nxt.append(merge2(*blocks[a], *blocks[a + 1]))
                if len(blocks) % 2:
                    nxt.append(blocks[-1])
                blocks = nxt
            top_v, top_i = merge2(top_v, top_i, *blocks[0])
            bound = _splat(top_v, 15)
            return top_v, top_i, bound

        def group_body(g, carry):
            qs = pl.ds(r0 + g * 16, 16)
            qx = xv[qs]
            qy = yv[qs]
            qz = zv[qs]
            qbx = _bf_round(qx)
            qby = _bf_round(qy)
            qbz = _bf_round(qz)
            qsq = qx * qx + qy * qy + qz * qz

            def row_body(l, carry2):
                bx = _splat(qbx, l)
                by = _splat(qby, l)
                bz = _splat(qbz, l)
                sq = _splat(qsq, l)

                zf = jnp.zeros((16,), jnp.float32)
                zi = jnp.zeros((16,), jnp.int32)
                fmask = lanes < 0

                def do_append(st2):
                    top_v, top_i, bound, pos, pdes, pms, pbase = st2
                    for u in range(8):
                        cu = plsc.all_reduce_population_count(pms[u])[0]
                        plsc.store_compressed(
                            bufd.at[pl.ds(pos, 16)], pdes[u], mask=pms[u]
                        )
                        plsc.store_compressed(
                            bufi.at[pl.ds(pos, 16)],
                            lanes + (pbase + u * 16),
                            mask=pms[u],
                        )
                        pos = pos + cu

                    def do_flush(st3):
                        tv, ti, _, p = st3
                        tv, ti, bnd = flush(tv, ti, p)
                        return tv, ti, bnd, jnp.int32(0)

                    return lax.cond(
                        pos >= 64,
                        do_flush,
                        lambda st3: st3,
                        (top_v, top_i, bound, pos),
                    )

                def chunk_body(j, st):
                    (top_v, top_i, bound, pos, panyc, pdes, pms, pbase) = st
                    base = j * 128
                    des, ms = [], []
                    any_m = None
                    for u in range(8):
                        s = pl.ds(base + u * 16, 16)
                        d = (
                            sq + ((bx * xm2[s] + by * ym2[s]) + bz * zm2[s])
                        ) + csqv[s]
                        de = jnp.where(d > _TH, _INF, d)
                        m = de < bound
                        des.append(de)
                        ms.append(m)
                        any_m = m if any_m is None else (any_m | m)
                    anyc = plsc.all_reduce_population_count(any_m)[0]

                    top_v, top_i, bound, pos = lax.cond(
                        panyc > 0,
                        do_append,
                        lambda st2: st2[:4],
                        (top_v, top_i, bound, pos, pdes, pms, pbase),
                    )
                    return (top_v, top_i, bound, pos, anyc, tuple(des), tuple(ms), base)

                init = (
                    jnp.full((16,), _INF, jnp.float32),
                    lanes,
                    jnp.full((16,), _INF, jnp.float32),
                    jnp.int32(0),
                    jnp.int32(0),
                    (zf,) * 8,
                    (fmask,) * 8,
                    jnp.int32(0),
                )
                (top_v, top_i, bound, pos, panyc, pdes, pms, pbase) = lax.fori_loop(
                    0, chunks // 8, chunk_body, init
                )
                top_v, top_i, bound, pos = lax.cond(
                    panyc > 0,
                    do_append,
                    lambda st2: st2[:4],
                    (top_v, top_i, bound, pos, pdes, pms, pbase),
                )
                top_v, top_i, bound = flush(top_v, top_i, pos)
                up = jnp.minimum(lanes + 1, 15)
                dn = jnp.maximum(lanes - 1, 0)
                nv = top_v.at[up].get(mode="promise_in_bounds")
                ni = top_i.at[up].get(mode="promise_in_bounds")
                pv = top_v.at[dn].get(mode="promise_in_bounds")
                pi = top_i.at[dn].get(mode="promise_in_bounds")
                swap_hi = (top_v == nv) & (top_i > ni)
                swap_lo = (top_v == pv) & (pi > top_i)
                top_i = jnp.where(
                    swap_hi, ni, jnp.where(swap_lo, pi, top_i)
                )
                obuf[pl.ds((g * 16 + l) * _K, _K)] = top_i
                return carry2

            lax.fori_loop(0, 16, row_body, 0)
            return carry

        lax.fori_loop(0, groups, group_body, 0)
        pltpu.sync_copy(
            obuf, out_hbm.at[pl.ds((batch * n + r0) * _K, rows_per_w * _K)]
        )

    return sc_knn


def kernel(coords):
    b, n, d = coords.shape
    xa = coords[:, :, 0].reshape(-1)
    ya = coords[:, :, 1].reshape(-1)
    za = coords[:, :, 2].reshape(-1)
    nn_idx = _make_sc_knn(b, n)(xa, ya, za).reshape(b, n, _K)
    center_idx = jnp.broadcast_to(
        jnp.arange(n, dtype=nn_idx.dtype)[None, :, None], (b, n, _K)
    )
    return jnp.stack((nn_idx, center_idx), axis=0)


# speedup vs baseline: 6.1562x; 1.0009x over previous
import functools

import jax
import jax.numpy as jnp
from jax import lax
from jax.experimental import pallas as pl
from jax.experimental.pallas import tpu as pltpu
from jax.experimental.pallas import tpu_sc as plsc

_K = 16
_TH = 0.5
_INF = float("inf")


def _bf_round(x):
    u = plsc.bitcast(x, jnp.uint32)
    r = (u + jnp.uint32(0x7FFF) + ((u >> 16) & jnp.uint32(1))) & jnp.uint32(
        0xFFFF0000
    )
    return plsc.bitcast(r, jnp.float32)


def _splat(v, lane):
    idx = jnp.full((16,), lane, jnp.int32)
    return v.at[idx].get(mode="promise_in_bounds", unique_indices=False)


def _make_sc_knn(b, n):
    rows_per_w = (b * n) // 32
    groups = rows_per_w // 16
    chunks = n // 16
    wpb = 32 // b
    mesh = plsc.VectorSubcoreMesh(core_axis_name="c", subcore_axis_name="s")

    @functools.partial(
        pl.kernel,
        out_type=jax.ShapeDtypeStruct((b * n * _K,), jnp.int32),
        mesh=mesh,
        compiler_params=pltpu.CompilerParams(needs_layout_passes=False),
        scratch_types=[
            pltpu.VMEM((n,), jnp.float32),
            pltpu.VMEM((n,), jnp.float32),
            pltpu.VMEM((n,), jnp.float32),
            pltpu.VMEM((n,), jnp.bfloat16),
            pltpu.VMEM((n,), jnp.bfloat16),
            pltpu.VMEM((n,), jnp.bfloat16),
            pltpu.VMEM((n,), jnp.float32),
            pltpu.VMEM((224,), jnp.float32),
            pltpu.VMEM((224,), jnp.int32),
            pltpu.VMEM((rows_per_w * _K,), jnp.int32),
        ],
    )
    def sc_knn(xa, ya, za, out_hbm, xv, yv, zv, xm2, ym2, zm2, csqv, bufd, bufi, obuf):
        wid = lax.axis_index("s") * 2 + lax.axis_index("c")
        batch = wid // wpb
        r0 = (wid % wpb) * rows_per_w

        pltpu.sync_copy(xa.at[pl.ds(batch * n, n)], xv)
        pltpu.sync_copy(ya.at[pl.ds(batch * n, n)], yv)
        pltpu.sync_copy(za.at[pl.ds(batch * n, n)], zv)

        def prep(i, carry):
            s0 = pl.ds(i * 32, 16)
            s1 = pl.ds(i * 32 + 16, 16)
            sw = pl.ds(i * 32, 32)
            cx0, cx1 = xv[s0], xv[s1]
            cy0, cy1 = yv[s0], yv[s1]
            cz0, cz1 = zv[s0], zv[s1]
            fmt = plsc.PackFormat.INTERLEAVED
            xm2[sw] = plsc.pack(_bf_round(cx0) * -2.0, _bf_round(cx1) * -2.0, format=fmt)
            ym2[sw] = plsc.pack(_bf_round(cy0) * -2.0, _bf_round(cy1) * -2.0, format=fmt)
            zm2[sw] = plsc.pack(_bf_round(cz0) * -2.0, _bf_round(cz1) * -2.0, format=fmt)
            csqv[s0] = cx0 * cx0 + cy0 * cy0 + cz0 * cz0
            csqv[s1] = cx1 * cx1 + cy1 * cy1 + cz1 * cz1
            return carry

        lax.fori_loop(0, chunks // 2, prep, 0)

        lanes = lax.iota(jnp.int32, 16)

        def merge2(av, ai, bv, bi):
            rv = lax.rev(bv, (0,))
            ri = lax.rev(bi, (0,))
            take_b = rv < av
            nv = jnp.where(take_b, rv, av)
            ni = jnp.where(take_b, ri, ai)
            return plsc.sort_key_val(nv, ni)

        def flush(top_v, top_i, pos):
            blocks = []
            for blk in range(12):
                bd = bufd[pl.ds(blk * 16, 16)]
                bi = bufi[pl.ds(blk * 16, 16)]
                valid = (lanes + blk * 16) < pos
                bd = jnp.where(valid, bd, _INF)
                blocks.append(plsc.sort_key_val(bd, bi))
            while len(blocks) > 1:
                nxt = []
                for a in range(0, len(blocks) - 1, 2):
                    nxt.append(merge2(*blocks[a], *blocks[a + 1]))
                if len(blocks) % 2:
                    nxt.append(blocks[-1])
                blocks = nxt
            top_v, top_i = merge2(top_v, top_i, *blocks[0])
            bound = _splat(top_v, 15)
            return top_v, top_i, bound

        def group_body(g, carry):
            qs = pl.ds(r0 + g * 16, 16)
            qx = xv[qs]
            qy = yv[qs]
            qz = zv[qs]
            qbx = _bf_round(qx)
            qby = _bf_round(qy)
            qbz = _bf_round(qz)
            qsq = qx * qx + qy * qy + qz * qz

            def row_body(l, carry2):
                bx = _splat(qbx, l)
                by = _splat(qby, l)
                bz = _splat(qbz, l)
                sq = _splat(qsq, l)

                zf = jnp.zeros((16,), jnp.float32)
                zi = jnp.zeros((16,), jnp.int32)
                fmask = lanes < 0

                def do_append(st2):
                    top_v, top_i, bound, pos, pdes, pms, pbase = st2
                    for u in range(8):
                        cu = plsc.all_reduce_population_count(pms[u])[0]
                        plsc.store_compressed(
                            bufd.at[pl.ds(pos, 16)], pdes[u], mask=pms[u]
                        )
                        plsc.store_compressed(
                            bufi.at[pl.ds(pos, 16)],
                            lanes + (pbase + u * 16),
                            mask=pms[u],
                        )
                        pos = pos + cu

                    def do_flush(st3):
                        tv, ti, _, p = st3
                        tv, ti, bnd = flush(tv, ti, p)
                        return tv, ti, bnd, jnp.int32(0)

                    return lax.cond(
                        pos >= 64,
                        do_flush,
                        lambda st3: st3,
                        (top_v, top_i, bound, pos),
                    )

                def chunk_body(j, st):
                    (top_v, top_i, bound, pos, panyc, pdes, pms, pbase) = st
                    base = j * 128
                    des, ms = [], []
                    any_m = None
                    fmt = plsc.PackFormat.INTERLEAVED
                    for w in range(4):
                        sw = pl.ds(base + w * 32, 32)
                        xs = plsc.unpack(xm2[sw], format=fmt)
                        ys = plsc.unpack(ym2[sw], format=fmt)
                        zs = plsc.unpack(zm2[sw], format=fmt)
                        for h in range(2):
                            u = w * 2 + h
                            s = pl.ds(base + u * 16, 16)
                            d = (
                                sq + ((bx * xs[h] + by * ys[h]) + bz * zs[h])
                            ) + csqv[s]
                            de = jnp.where(d > _TH, _INF, d)
                            m = de < bound
                            des.append(de)
                            ms.append(m)
                            any_m = m if any_m is None else (any_m | m)
                    anyc = plsc.all_reduce_population_count(any_m)[0]

                    top_v, top_i, bound, pos = lax.cond(
                        panyc > 0,
                        do_append,
                        lambda st2: st2[:4],
                        (top_v, top_i, bound, pos, pdes, pms, pbase),
                    )
                    return (top_v, top_i, bound, pos, anyc, tuple(des), tuple(ms), base)

                init = (
                    jnp.full((16,), _INF, jnp.float32),
                    lanes,
                    jnp.full((16,), _INF, jnp.float32),
                    jnp.int32(0),
                    jnp.int32(0),
                    (zf,) * 8,
                    (fmask,) * 8,
                    jnp.int32(0),
                )
                (top_v, top_i, bound, pos, panyc, pdes, pms, pbase) = lax.fori_loop(
                    0, chunks // 8, chunk_body, init
                )
                top_v, top_i, bound, pos = lax.cond(
                    panyc > 0,
                    do_append,
                    lambda st2: st2[:4],
                    (top_v, top_i, bound, pos, pdes, pms, pbase),
                )
                top_v, top_i, bound = flush(top_v, top_i, pos)
                up = jnp.minimum(lanes + 1, 15)
                dn = jnp.maximum(lanes - 1, 0)
                nv = top_v.at[up].get(mode="promise_in_bounds")
                ni = top_i.at[up].get(mode="promise_in_bounds")
                pv = top_v.at[dn].get(mode="promise_in_bounds")
                pi = top_i.at[dn].get(mode="promise_in_bounds")
                swap_hi = (top_v == nv) & (top_i > ni)
                swap_lo = (top_v == pv) & (pi > top_i)
                top_i = jnp.where(
                    swap_hi, ni, jnp.where(swap_lo, pi, top_i)
                )
                obuf[pl.ds((g * 16 + l) * _K, _K)] = top_i
                return carry2

            lax.fori_loop(0, 16, row_body, 0)
            return carry

        lax.fori_loop(0, groups, group_body, 0)
        pltpu.sync_copy(
            obuf, out_hbm.at[pl.ds((batch * n + r0) * _K, rows_per_w * _K)]
        )

    return sc_knn


def kernel(coords):
    b, n, d = coords.shape
    xa = coords[:, :, 0].reshape(-1)
    ya = coords[:, :, 1].reshape(-1)
    za = coords[:, :, 2].reshape(-1)
    nn_idx = _make_sc_knn(b, n)(xa, ya, za).reshape(b, n, _K)
    center_idx = jnp.broadcast_to(
        jnp.arange(n, dtype=nn_idx.dtype)[None, :, None], (b, n, _K)
    )
    return jnp.stack((nn_idx, center_idx), axis=0)
